# baseline (speedup 1.0000x reference)
import math

import jax
import jax.numpy as jnp
from jax.experimental import pallas as pl
from jax.experimental.pallas import tpu as pltpu

_N = 8
_D = 32
_LAYERS = 2
_BN_EPS = 1e-5
_LN_EPS = 1e-3
_GB = 32
_M = _GB * _N
_NG = 16
_C = _NG * _M

_CF0, _CF1, _CAT = 2, 18, 18


def _g8(x, lanemod, op, k):
    n = x.shape[-1]
    a = pltpu.roll(x, n - k, axis=1)
    b = pltpu.roll(x, _N - k, axis=1)
    return op(x, jnp.where(lanemod < _N - k, a, b))


def _split_dot(a, b_ref):
    f32 = jnp.float32
    a_hi = a.astype(jnp.bfloat16).astype(f32)
    a_lo = a - a_hi
    return (jnp.dot(a_hi, b_ref[...], preferred_element_type=f32)
            + jnp.dot(a_lo, b_ref[...], preferred_element_type=f32))


def _gcn_kernel(featT_ref, adjT_ref, w_ref, p_ref, mask_ref, pool_ref,
                natt_ref, pred_ref):
    f32 = jnp.float32
    w0T = w_ref[0]
    w1T = w_ref[1]
    wfT = w_ref[_LAYERS]
    lb0T = p_ref[:, 0:1]
    lb1T = p_ref[:, 1:2]
    hbT = p_ref[:, 2:3]
    a1b = p_ref[0:1, 3:4]
    a2b = p_ref[1:2, 3:4]
    pb = p_ref[2:3, 3:4]
    pwT = p_ref[0:16, 4:5]

    featT = jnp.transpose(featT_ref[...].reshape(_C, _D), (1, 0))
    adjT = jnp.transpose(adjT_ref[...].reshape(_C, _N), (1, 0))

    attsT = jnp.dot(wfT, featT, preferred_element_type=f32)
    na = attsT[0:1, :] + a1b
    ea = attsT[1:2, :] + a2b

    lanemod = jax.lax.broadcasted_iota(jnp.int32, (1, _C), 1) % _N
    mx = na
    for k in (1, 2, 4):
        mx = _g8(mx, lanemod, jnp.maximum, k)
    e = jnp.exp(na - mx)
    s = e
    for k in (1, 2, 4):
        s = _g8(s, lanemod, jnp.add, k)
    natt_ref[0, 0:1, :] = e / s

    z = ea * adjT
    bdT = [pltpu.repeat(z[:, u * _M:(u + 1) * _M], _GB, 0) * mask_ref[...]
           for u in range(_NG)]

    def bd_apply(x, bias):
        parts = [jnp.dot(x[:, u * _M:(u + 1) * _M], bdT[u],
                         preferred_element_type=f32) for u in range(_NG)]
        return jnp.concatenate(parts, axis=1) + bias

    hid = na * featT
    s1 = jnp.dot(w0T, hid, preferred_element_type=f32)
    hid = jnp.tanh(bd_apply(s1, lb0T)) + hid
    s2 = jnp.dot(w1T, hid, preferred_element_type=f32)
    hid = jnp.tanh(bd_apply(s2, lb1T)) + hid

    heads = jnp.dot(wfT, hid, preferred_element_type=f32)
    ho = bd_apply(heads, hbT)
    agg = ho[_CF0:_CF1, :]
    att = jnp.tanh(ho[_CAT:_CAT + 1, :])

    wagg = att * agg
    pf = jnp.concatenate(
        [_split_dot(wagg[:, u * _M:(u + 1) * _M], pool_ref)
         for u in range(_NG)], axis=1)
    mu = jnp.mean(pf, axis=0, keepdims=True)
    var = jnp.mean((pf - mu) ** 2, axis=0, keepdims=True)
    pfln = jnp.tanh((pf - mu) * jax.lax.rsqrt(var + _LN_EPS))

    aa = jnp.concatenate(
        [jnp.dot(att[:, u * _M:(u + 1) * _M], bdT[u],
                 preferred_element_type=f32) for u in range(_NG)], axis=1)
    q = att * aa
    pm = jnp.tanh(jnp.concatenate(
        [_split_dot(q[:, u * _M:(u + 1) * _M], pool_ref)
         for u in range(_NG)], axis=1))

    sp = jnp.sum(pfln * pwT, axis=0, keepdims=True)
    pred_ref[0, 0:1, :] = pm * sp + pb


def _pack(a1w, a1b, a2w, a2b, lw, lb, fw, fb, aw, ab, pw, pb):
    d = _D
    scale = 1.0 / math.sqrt(1.0 + _BN_EPS)
    wf = jnp.zeros((d, d), jnp.float32)
    wf = wf.at[:, 0].set(a1w[:, 0])
    wf = wf.at[:, 1].set(a2w[:, 0])
    wf = wf.at[:, _CF0:_CF1].set(fw * scale)
    wf = wf.at[:, _CAT].set(aw[:, 0] * scale)
    w_slab = jnp.stack([lw[0].T, lw[1].T, wf.T], axis=0)

    p_slab = jnp.zeros((d, 5), jnp.float32)
    p_slab = p_slab.at[:, 0].set(lb[0, 0, :])
    p_slab = p_slab.at[:, 1].set(lb[1, 0, :])
    hbias = jnp.zeros((d,), jnp.float32)
    hbias = hbias.at[_CF0:_CF1].set(fb[0] * scale)
    hbias = hbias.at[_CAT].set(ab[0, 0] * scale)
    p_slab = p_slab.at[:, 2].set(hbias)
    p_slab = p_slab.at[0, 3].set(a1b[0, 0])
    p_slab = p_slab.at[1, 3].set(a2b[0, 0])
    p_slab = p_slab.at[2, 3].set(pb[0, 0])
    p_slab = p_slab.at[0:16, 4].set(pw[:, 0])
    return w_slab, p_slab


def kernel(adj, feat, a1w, a1b, a2w, a2b, lw, lb, fw, fb, aw, ab, pw, pb):
    b = adj.shape[0]
    w_slab, p_slab = _pack(a1w, a1b, a2w, a2b, lw, lb, fw, fb, aw, ab, pw, pb)


    mask = jnp.kron(jnp.eye(_GB, dtype=jnp.float32),
                    jnp.ones((_N, _N), jnp.float32))
    pool = jnp.kron(jnp.eye(_GB, dtype=jnp.float32),
                    jnp.ones((_N, 1), jnp.float32))

    nsteps = b // (_NG * _GB)
    natt_out, pred_out = pl.pallas_call(
        _gcn_kernel,
        out_shape=(
            jax.ShapeDtypeStruct((nsteps, 1, _C), jnp.float32),
            jax.ShapeDtypeStruct((nsteps, 1, _NG * _GB), jnp.float32),
        ),
        grid=(nsteps,),
        in_specs=[
            pl.BlockSpec((_C // _N, _N, _D), lambda i: (i, 0, 0)),
            pl.BlockSpec((_C // _N, _N, _N), lambda i: (i, 0, 0)),
            pl.BlockSpec((_LAYERS + 1, _D, _D), lambda i: (0, 0, 0)),
            pl.BlockSpec((_D, 5), lambda i: (0, 0)),
            pl.BlockSpec((_M, _M), lambda i: (0, 0)),
            pl.BlockSpec((_M, _GB), lambda i: (0, 0)),
        ],
        out_specs=(
            pl.BlockSpec((1, 1, _C), lambda i: (i, 0, 0)),
            pl.BlockSpec((1, 1, _NG * _GB), lambda i: (i, 0, 0)),
        ),
        compiler_params=pltpu.CompilerParams(
            dimension_semantics=("parallel",),
        ),
    )(feat, adj, w_slab, p_slab, mask, pool)

    return pred_out.reshape(b, 1), natt_out.reshape(b, _N)


# speedup vs baseline: 1.1472x; 1.1472x over previous
import math

import jax
import jax.numpy as jnp
from jax.experimental import pallas as pl
from jax.experimental.pallas import tpu as pltpu

_N = 8
_D = 32
_LAYERS = 2
_BN_EPS = 1e-5
_LN_EPS = 1e-3
_GB = 32
_M = _GB * _N
_NG = 16
_C = _NG * _M

_CF0, _CF1, _CAT = 2, 18, 18


def _g8(x, lanemod, op, k):
    n = x.shape[-1]
    a = pltpu.roll(x, n - k, axis=1)
    b = pltpu.roll(x, _N - k, axis=1)
    return op(x, jnp.where(lanemod < _N - k, a, b))


def _split_dot(a, b_ref):
    f32 = jnp.float32
    a_hi = a.astype(jnp.bfloat16).astype(f32)
    a_lo = a - a_hi
    return (jnp.dot(a_hi, b_ref[...], preferred_element_type=f32)
            + jnp.dot(a_lo, b_ref[...], preferred_element_type=f32))


def _gcn_kernel(featT_ref, adjT_ref, w_ref, p_ref, mask_ref, pool_ref,
                natt_ref, pred_ref):
    f32 = jnp.float32
    w0T = w_ref[0]
    w1T = w_ref[1]
    wfT = w_ref[_LAYERS]
    lb0T = p_ref[:, 0:1]
    lb1T = p_ref[:, 1:2]
    hbT = p_ref[:, 2:3]
    a1b = p_ref[0:1, 3:4]
    a2b = p_ref[1:2, 3:4]
    pb = p_ref[2:3, 3:4]
    pwT = p_ref[0:16, 4:5]

    featT = jnp.transpose(featT_ref[...], (1, 0))
    adjT = jnp.transpose(adjT_ref[...].reshape(_C, _N), (1, 0))

    attsT = jnp.dot(wfT[0:8, :], featT,
                    preferred_element_type=f32)
    na = attsT[0:1, :] + a1b
    ea = attsT[1:2, :] + a2b

    lanemod = jax.lax.broadcasted_iota(jnp.int32, (1, _C), 1) % _N
    mx = na
    for k in (1, 2, 4):
        mx = _g8(mx, lanemod, jnp.maximum, k)
    e = jnp.exp(na - mx)
    s = e
    for k in (1, 2, 4):
        s = _g8(s, lanemod, jnp.add, k)
    natt_ref[0, 0:1, :] = e / s

    z = ea * adjT
    bdT = [pltpu.repeat(z[:, u * _M:(u + 1) * _M], _GB, 0) * mask_ref[...]
           for u in range(_NG)]

    def bd_apply(x, bias):
        parts = [jnp.dot(x[:, u * _M:(u + 1) * _M], bdT[u],
                         preferred_element_type=f32) for u in range(_NG)]
        return jnp.concatenate(parts, axis=1) + bias

    hid = na * featT
    s1 = jnp.dot(w0T, hid, preferred_element_type=f32)
    hid = jnp.tanh(bd_apply(s1, lb0T)) + hid
    s2 = jnp.dot(w1T, hid, preferred_element_type=f32)
    hid = jnp.tanh(bd_apply(s2, lb1T)) + hid

    heads = jnp.dot(wfT, hid, preferred_element_type=f32)
    ho_parts, aa_parts = [], []
    for u in range(_NG):
        ho_u = jnp.dot(heads[:, u * _M:(u + 1) * _M], bdT[u],
                       preferred_element_type=f32) + hbT
        att_u = jnp.tanh(ho_u[_CAT:_CAT + 1, :])
        aa_parts.append(jnp.dot(att_u, bdT[u], preferred_element_type=f32))
        ho_parts.append(ho_u)
    ho = jnp.concatenate(ho_parts, axis=1)
    agg = ho[_CF0:_CF1, :]
    att = jnp.tanh(ho[_CAT:_CAT + 1, :])

    wagg = att * agg
    pf = jnp.concatenate(
        [_split_dot(wagg[:, u * _M:(u + 1) * _M], pool_ref)
         for u in range(_NG)], axis=1)
    mu = jnp.mean(pf, axis=0, keepdims=True)
    var = jnp.mean((pf - mu) ** 2, axis=0, keepdims=True)
    pfln = jnp.tanh((pf - mu) * jax.lax.rsqrt(var + _LN_EPS))

    aa = jnp.concatenate(aa_parts, axis=1)
    q = att * aa
    pm = jnp.tanh(jnp.concatenate(
        [_split_dot(q[:, u * _M:(u + 1) * _M], pool_ref)
         for u in range(_NG)], axis=1))

    sp = jnp.sum(pfln * pwT, axis=0, keepdims=True)
    pred_ref[0, 0:1, :] = pm * sp + pb


def _pack(a1w, a1b, a2w, a2b, lw, lb, fw, fb, aw, ab, pw, pb):
    d = _D
    scale = 1.0 / math.sqrt(1.0 + _BN_EPS)
    wf = jnp.zeros((d, d), jnp.float32)
    wf = wf.at[:, 0].set(a1w[:, 0])
    wf = wf.at[:, 1].set(a2w[:, 0])
    wf = wf.at[:, _CF0:_CF1].set(fw * scale)
    wf = wf.at[:, _CAT].set(aw[:, 0] * scale)
    w_slab = jnp.stack([lw[0].T, lw[1].T, wf.T], axis=0)

    p_slab = jnp.zeros((d, 5), jnp.float32)
    p_slab = p_slab.at[:, 0].set(lb[0, 0, :])
    p_slab = p_slab.at[:, 1].set(lb[1, 0, :])
    hbias = jnp.zeros((d,), jnp.float32)
    hbias = hbias.at[_CF0:_CF1].set(fb[0] * scale)
    hbias = hbias.at[_CAT].set(ab[0, 0] * scale)
    p_slab = p_slab.at[:, 2].set(hbias)
    p_slab = p_slab.at[0, 3].set(a1b[0, 0])
    p_slab = p_slab.at[1, 3].set(a2b[0, 0])
    p_slab = p_slab.at[2, 3].set(pb[0, 0])
    p_slab = p_slab.at[0:16, 4].set(pw[:, 0])
    return w_slab, p_slab


def kernel(adj, feat, a1w, a1b, a2w, a2b, lw, lb, fw, fb, aw, ab, pw, pb):
    b = adj.shape[0]
    w_slab, p_slab = _pack(a1w, a1b, a2w, a2b, lw, lb, fw, fb, aw, ab, pw, pb)


    mask = jnp.kron(jnp.eye(_GB, dtype=jnp.float32),
                    jnp.ones((_N, _N), jnp.float32))
    pool = jnp.kron(jnp.eye(_GB, dtype=jnp.float32),
                    jnp.ones((_N, 1), jnp.float32))

    nsteps = b // (_NG * _GB)
    natt_out, pred_out = pl.pallas_call(
        _gcn_kernel,
        out_shape=(
            jax.ShapeDtypeStruct((nsteps, 1, _C), jnp.float32),
            jax.ShapeDtypeStruct((nsteps, 1, _NG * _GB), jnp.float32),
        ),
        grid=(nsteps,),
        in_specs=[
            pl.BlockSpec((_C, _D), lambda i: (i, 0)),
            pl.BlockSpec((_C // _N, _N, _N), lambda i: (i, 0, 0)),
            pl.BlockSpec((_LAYERS + 1, _D, _D), lambda i: (0, 0, 0)),
            pl.BlockSpec((_D, 5), lambda i: (0, 0)),
            pl.BlockSpec((_M, _M), lambda i: (0, 0)),
            pl.BlockSpec((_M, _GB), lambda i: (0, 0)),
        ],
        out_specs=(
            pl.BlockSpec((1, 1, _C), lambda i: (i, 0, 0)),
            pl.BlockSpec((1, 1, _NG * _GB), lambda i: (i, 0, 0)),
        ),
        compiler_params=pltpu.CompilerParams(
            dimension_semantics=("parallel",),
        ),
    )(feat.reshape(b * _N, _D), adj, w_slab, p_slab, mask, pool)

    return pred_out.reshape(b, 1), natt_out.reshape(b, _N)


# speedup vs baseline: 1.2917x; 1.1259x over previous
import math

import numpy as np

import jax
import jax.numpy as jnp
from jax.experimental import pallas as pl
from jax.experimental.pallas import tpu as pltpu

_N = 8
_D = 32
_LAYERS = 2
_BN_EPS = 1e-5
_LN_EPS = 1e-3
_GB = 32
_M = _GB * _N
_NG = 16
_C = _NG * _M

_CF0, _CF1, _CAT = 2, 18, 18

_MASK_NP = np.kron(np.eye(_GB, dtype=np.float32), np.ones((_N, _N), np.float32))
_POOL_NP = np.kron(np.eye(_GB, dtype=np.float32), np.ones((_N, 1), np.float32))


def _g8(x, lanemod, op, k):
    n = x.shape[-1]
    a = pltpu.roll(x, n - k, axis=1)
    b = pltpu.roll(x, _N - k, axis=1)
    return op(x, jnp.where(lanemod < _N - k, a, b))


def _split_dot(a, b_ref):
    f32 = jnp.float32
    a_hi = a.astype(jnp.bfloat16).astype(f32)
    a_lo = a - a_hi
    return (jnp.dot(a_hi, b_ref[...], preferred_element_type=f32)
            + jnp.dot(a_lo, b_ref[...], preferred_element_type=f32))


def _gcn_kernel(featT_ref, adjT_ref, w_ref, p_ref, mask_ref, pool_ref,
                natt_ref, pred_ref):
    f32 = jnp.float32
    w0T = w_ref[0]
    w1T = w_ref[1]
    wfT = w_ref[_LAYERS]
    lb0T = p_ref[:, 0:1]
    lb1T = p_ref[:, 1:2]
    hbT = p_ref[:, 2:3]
    a1b = p_ref[0:1, 3:4]
    a2b = p_ref[1:2, 3:4]
    pb = p_ref[2:3, 3:4]
    pwT = p_ref[0:16, 4:5]

    featT = jnp.transpose(featT_ref[...], (1, 0))
    adjT = jnp.transpose(adjT_ref[...].reshape(_C, _N), (1, 0))

    attsT = jnp.dot(wfT[0:8, :], featT,
                    preferred_element_type=f32)
    na = attsT[0:1, :] + a1b
    ea = attsT[1:2, :] + a2b

    lanemod = jax.lax.broadcasted_iota(jnp.int32, (1, _C), 1) % _N
    mx = na
    for k in (1, 2, 4):
        mx = _g8(mx, lanemod, jnp.maximum, k)
    e = jnp.exp(na - mx)
    s = e
    for k in (1, 2, 4):
        s = _g8(s, lanemod, jnp.add, k)
    natt_ref[0, 0:1, :] = e / s

    z = ea * adjT
    bdT = [pltpu.repeat(z[:, u * _M:(u + 1) * _M], _GB, 0) * mask_ref[...]
           for u in range(_NG)]

    def bd_apply(x, bias):
        parts = [jnp.dot(x[:, u * _M:(u + 1) * _M], bdT[u],
                         preferred_element_type=f32) for u in range(_NG)]
        return jnp.concatenate(parts, axis=1) + bias

    hid = na * featT
    s1 = jnp.dot(w0T, hid, preferred_element_type=f32)
    hid = jnp.tanh(bd_apply(s1, lb0T)) + hid
    s2 = jnp.dot(w1T, hid, preferred_element_type=f32)
    hid = jnp.tanh(bd_apply(s2, lb1T)) + hid

    heads = jnp.dot(wfT, hid, preferred_element_type=f32)
    ho_parts, aa_parts = [], []
    for u in range(_NG):
        ho_u = jnp.dot(heads[:, u * _M:(u + 1) * _M], bdT[u],
                       preferred_element_type=f32) + hbT
        att_u = jnp.tanh(ho_u[_CAT:_CAT + 1, :])
        aa_parts.append(jnp.dot(att_u, bdT[u], preferred_element_type=f32))
        ho_parts.append(ho_u)
    ho = jnp.concatenate(ho_parts, axis=1)
    agg = ho[_CF0:_CF1, :]
    att = jnp.tanh(ho[_CAT:_CAT + 1, :])

    wagg = att * agg
    pf = jnp.concatenate(
        [_split_dot(wagg[:, u * _M:(u + 1) * _M], pool_ref)
         for u in range(_NG)], axis=1)
    mu = jnp.mean(pf, axis=0, keepdims=True)
    var = jnp.mean((pf - mu) ** 2, axis=0, keepdims=True)
    pfln = jnp.tanh((pf - mu) * jax.lax.rsqrt(var + _LN_EPS))

    aa = jnp.concatenate(aa_parts, axis=1)
    q = att * aa
    pm = jnp.tanh(jnp.concatenate(
        [_split_dot(q[:, u * _M:(u + 1) * _M], pool_ref)
         for u in range(_NG)], axis=1))

    sp = jnp.sum(pfln * pwT, axis=0, keepdims=True)
    pred_ref[0, 0:1, :] = pm * sp + pb


def _pack(a1w, a1b, a2w, a2b, lw, lb, fw, fb, aw, ab, pw, pb):
    f32 = jnp.float32
    scale = 1.0 / math.sqrt(1.0 + _BN_EPS)
    z = lambda r: jnp.zeros((r, 1), f32)
    wfT = jnp.concatenate(
        [a1w.T, a2w.T, fw.T * scale, aw.T * scale,
         jnp.zeros((_D - _CAT - 1, _D), f32)], axis=0)
    w_slab = jnp.stack([lw[0].T, lw[1].T, wfT], axis=0)

    c2 = jnp.concatenate([z(_CF0), fb.T * scale, ab * scale,
                          z(_D - _CAT - 1)], axis=0)
    c3 = jnp.concatenate([a1b, a2b, pb, z(_D - 3)], axis=0)
    c4 = jnp.concatenate([pw, z(_D - 16)], axis=0)
    p_slab = jnp.concatenate(
        [lb[0].T, lb[1].T, c2, c3, c4], axis=1)
    return w_slab, p_slab


def kernel(adj, feat, a1w, a1b, a2w, a2b, lw, lb, fw, fb, aw, ab, pw, pb):
    b = adj.shape[0]
    w_slab, p_slab = _pack(a1w, a1b, a2w, a2b, lw, lb, fw, fb, aw, ab, pw, pb)


    mask = jnp.asarray(_MASK_NP)
    pool = jnp.asarray(_POOL_NP)

    nsteps = b // (_NG * _GB)
    natt_out, pred_out = pl.pallas_call(
        _gcn_kernel,
        out_shape=(
            jax.ShapeDtypeStruct((nsteps, 1, _C), jnp.float32),
            jax.ShapeDtypeStruct((nsteps, 1, _NG * _GB), jnp.float32),
        ),
        grid=(nsteps,),
        in_specs=[
            pl.BlockSpec((_C, _D), lambda i: (i, 0)),
            pl.BlockSpec((_C // _N, _N, _N), lambda i: (i, 0, 0)),
            pl.BlockSpec((_LAYERS + 1, _D, _D), lambda i: (0, 0, 0)),
            pl.BlockSpec((_D, 5), lambda i: (0, 0)),
            pl.BlockSpec((_M, _M), lambda i: (0, 0)),
            pl.BlockSpec((_M, _GB), lambda i: (0, 0)),
        ],
        out_specs=(
            pl.BlockSpec((1, 1, _C), lambda i: (i, 0, 0)),
            pl.BlockSpec((1, 1, _NG * _GB), lambda i: (i, 0, 0)),
        ),
        compiler_params=pltpu.CompilerParams(
            dimension_semantics=("parallel",),
        ),
    )(feat.reshape(b * _N, _D), adj, w_slab, p_slab, mask, pool)

    return pred_out.reshape(b, 1), natt_out.reshape(b, _N)


# speedup vs baseline: 1.3491x; 1.0445x over previous
import math

import numpy as np

import jax
import jax.numpy as jnp
from jax.experimental import pallas as pl
from jax.experimental.pallas import tpu as pltpu

_N = 8
_D = 32
_LAYERS = 2
_BN_EPS = 1e-5
_LN_EPS = 1e-3
_GB = 32
_M = _GB * _N
_NG = 32
_C = _NG * _M

_CF0, _CF1, _CAT = 2, 18, 18

_MASK_NP = np.kron(np.eye(_GB, dtype=np.float32), np.ones((_N, _N), np.float32))
_POOL_NP = np.kron(np.eye(_GB, dtype=np.float32), np.ones((_N, 1), np.float32))


def _g8(x, lanemod, op, k):
    n = x.shape[-1]
    a = pltpu.roll(x, n - k, axis=1)
    b = pltpu.roll(x, _N - k, axis=1)
    return op(x, jnp.where(lanemod < _N - k, a, b))


def _split_dot(a, b_ref):
    f32 = jnp.float32
    a_hi = a.astype(jnp.bfloat16).astype(f32)
    a_lo = a - a_hi
    return (jnp.dot(a_hi, b_ref[...], preferred_element_type=f32)
            + jnp.dot(a_lo, b_ref[...], preferred_element_type=f32))


def _gcn_kernel(featT_ref, adjT_ref, w_ref, p_ref, mask_ref, pool_ref,
                natt_ref, pred_ref):
    f32 = jnp.float32
    w0T = w_ref[0]
    w1T = w_ref[1]
    wfT = w_ref[_LAYERS]
    lb0T = p_ref[:, 0:1]
    lb1T = p_ref[:, 1:2]
    hbT = p_ref[:, 2:3]
    a1b = p_ref[0:1, 3:4]
    a2b = p_ref[1:2, 3:4]
    pb = p_ref[2:3, 3:4]
    pwT = p_ref[0:16, 4:5]

    featT = jnp.transpose(featT_ref[...], (1, 0))
    adjT = jnp.transpose(adjT_ref[...].reshape(_C, _N), (1, 0))

    attsT = jnp.dot(wfT[0:8, :], featT,
                    preferred_element_type=f32)
    na = attsT[0:1, :] + a1b
    ea = attsT[1:2, :] + a2b

    lanemod = jax.lax.broadcasted_iota(jnp.int32, (1, _C), 1) % _N
    mx = na
    for k in (1, 2, 4):
        mx = _g8(mx, lanemod, jnp.maximum, k)
    e = jnp.exp(na - mx)
    s = e
    for k in (1, 2, 4):
        s = _g8(s, lanemod, jnp.add, k)
    natt_ref[0, 0:1, :] = e / s

    z = ea * adjT
    bdT = [pltpu.repeat(z[:, u * _M:(u + 1) * _M], _GB, 0) * mask_ref[...]
           for u in range(_NG)]

    def bd_apply(x, bias):
        parts = [jnp.dot(x[:, u * _M:(u + 1) * _M], bdT[u],
                         preferred_element_type=f32) for u in range(_NG)]
        return jnp.concatenate(parts, axis=1) + bias

    hid = na * featT
    s1 = jnp.dot(w0T, hid, preferred_element_type=f32)
    hid = jnp.tanh(bd_apply(s1, lb0T)) + hid
    s2 = jnp.dot(w1T, hid, preferred_element_type=f32)
    hid = jnp.tanh(bd_apply(s2, lb1T)) + hid

    heads = jnp.dot(wfT, hid, preferred_element_type=f32)
    ho_parts, aa_parts = [], []
    for u in range(_NG):
        ho_u = jnp.dot(heads[:, u * _M:(u + 1) * _M], bdT[u],
                       preferred_element_type=f32) + hbT
        att_u = jnp.tanh(ho_u[_CAT:_CAT + 1, :])
        aa_parts.append(jnp.dot(att_u, bdT[u], preferred_element_type=f32))
        ho_parts.append(ho_u)
    ho = jnp.concatenate(ho_parts, axis=1)
    agg = ho[_CF0:_CF1, :]
    att = jnp.tanh(ho[_CAT:_CAT + 1, :])

    wagg = att * agg
    pf = jnp.concatenate(
        [_split_dot(wagg[:, u * _M:(u + 1) * _M], pool_ref)
         for u in range(_NG)], axis=1)
    mu = jnp.mean(pf, axis=0, keepdims=True)
    var = jnp.mean((pf - mu) ** 2, axis=0, keepdims=True)
    pfln = jnp.tanh((pf - mu) * jax.lax.rsqrt(var + _LN_EPS))

    aa = jnp.concatenate(aa_parts, axis=1)
    q = att * aa
    pm = jnp.tanh(jnp.concatenate(
        [_split_dot(q[:, u * _M:(u + 1) * _M], pool_ref)
         for u in range(_NG)], axis=1))

    sp = jnp.sum(pfln * pwT, axis=0, keepdims=True)
    pred_ref[0, 0:1, :] = pm * sp + pb


def _pack(a1w, a1b, a2w, a2b, lw, lb, fw, fb, aw, ab, pw, pb):
    f32 = jnp.float32
    scale = 1.0 / math.sqrt(1.0 + _BN_EPS)
    z = lambda r: jnp.zeros((r, 1), f32)
    wfT = jnp.concatenate(
        [a1w.T, a2w.T, fw.T * scale, aw.T * scale,
         jnp.zeros((_D - _CAT - 1, _D), f32)], axis=0)
    w_slab = jnp.stack([lw[0].T, lw[1].T, wfT], axis=0)

    c2 = jnp.concatenate([z(_CF0), fb.T * scale, ab * scale,
                          z(_D - _CAT - 1)], axis=0)
    c3 = jnp.concatenate([a1b, a2b, pb, z(_D - 3)], axis=0)
    c4 = jnp.concatenate([pw, z(_D - 16)], axis=0)
    p_slab = jnp.concatenate(
        [lb[0].T, lb[1].T, c2, c3, c4], axis=1)
    return w_slab, p_slab


def kernel(adj, feat, a1w, a1b, a2w, a2b, lw, lb, fw, fb, aw, ab, pw, pb):
    b = adj.shape[0]
    w_slab, p_slab = _pack(a1w, a1b, a2w, a2b, lw, lb, fw, fb, aw, ab, pw, pb)


    mask = jnp.asarray(_MASK_NP)
    pool = jnp.asarray(_POOL_NP)

    nsteps = b // (_NG * _GB)
    natt_out, pred_out = pl.pallas_call(
        _gcn_kernel,
        out_shape=(
            jax.ShapeDtypeStruct((nsteps, 1, _C), jnp.float32),
            jax.ShapeDtypeStruct((nsteps, 1, _NG * _GB), jnp.float32),
        ),
        grid=(nsteps,),
        in_specs=[
            pl.BlockSpec((_C, _D), lambda i: (i, 0)),
            pl.BlockSpec((_C // _N, _N, _N), lambda i: (i, 0, 0)),
            pl.BlockSpec((_LAYERS + 1, _D, _D), lambda i: (0, 0, 0)),
            pl.BlockSpec((_D, 5), lambda i: (0, 0)),
            pl.BlockSpec((_M, _M), lambda i: (0, 0)),
            pl.BlockSpec((_M, _GB), lambda i: (0, 0)),
        ],
        out_specs=(
            pl.BlockSpec((1, 1, _C), lambda i: (i, 0, 0)),
            pl.BlockSpec((1, 1, _NG * _GB), lambda i: (i, 0, 0)),
        ),
        compiler_params=pltpu.CompilerParams(
            dimension_semantics=("parallel",),
        ),
    )(feat.reshape(b * _N, _D), adj, w_slab, p_slab, mask, pool)

    return pred_out.reshape(b, 1), natt_out.reshape(b, _N)
